# Initial kernel scaffold; baseline (speedup 1.0000x reference)
import jax
import jax.numpy as jnp
from jax.experimental import pallas as pl


def kernel(id, sector, stage, region, deal_size, revenue_multiple, growth_rate, profitability, team_experience, market_size, deal_table, sector_table, stage_table, region_table, W1, b1, g1, beta1, W2, b2, g2, beta2):
    raise NotImplementedError("write your pallas kernel here")


# speedup vs baseline: 1.0437x; 1.0437x over previous
import functools

import jax
import jax.numpy as jnp
from jax import lax
from jax.experimental import pallas as pl
from jax.experimental.pallas import tpu as pltpu
from jax.experimental.pallas import tpu_sc as plsc

B = 4096
EMB = 64
H1 = 256
H2 = 128
NW = 32
BPW = B // NW
CH = 512
NCH = B // CH
EPS = 1e-5


def _gather_deal(deal_table, idx):
    mesh = plsc.VectorSubcoreMesh(core_axis_name="c", subcore_axis_name="s")

    @functools.partial(
        pl.kernel,
        mesh=mesh,
        compiler_params=pltpu.CompilerParams(use_tc_tiling_on_sc=False),
        out_type=jax.ShapeDtypeStruct((B, EMB), jnp.float32),
        scratch_types=[
            pltpu.VMEM((BPW,), jnp.int32),
            pltpu.VMEM((BPW, EMB), jnp.float32),
            pltpu.SemaphoreType.DMA,
        ],
    )
    def gk(table_hbm, idx_hbm, out_hbm, idx_v, rows_v, sem):
        wid = lax.axis_index("s") * 2 + lax.axis_index("c")
        base = wid * BPW
        pltpu.sync_copy(idx_hbm.at[pl.ds(base, BPW)], idx_v)
        pltpu.async_copy(table_hbm.at[idx_v], rows_v, sem).wait()
        pltpu.sync_copy(rows_v, out_hbm.at[pl.ds(base, BPW)])

    return gk(deal_table, idx)


def _mlp_body(idemb, idx3, num, stab, ttab, rtab, w1, b1, g1, be1,
              w2, b2, g2, be2, out, h1b, h2b, s1, q1, s2, q2):
    p = pl.program_id(0)
    j = pl.program_id(1)
    base = j * CH

    @pl.when(p == 0)
    def _phase0():
        sec = idx3[:, 0:1]
        stg = idx3[:, 1:2]
        reg = idx3[:, 2:3]
        oh_s = (lax.broadcasted_iota(jnp.int32, (CH, 32), 1) == sec
                ).astype(jnp.float32)
        oh_t = (lax.broadcasted_iota(jnp.int32, (CH, 16), 1) == stg
                ).astype(jnp.float32)
        oh_r = (lax.broadcasted_iota(jnp.int32, (CH, 32), 1) == reg
                ).astype(jnp.float32)
        ws = jnp.dot(stab[...], w1[64:80, :], preferred_element_type=jnp.float32)
        wt = jnp.dot(ttab[...], w1[80:96, :], preferred_element_type=jnp.float32)
        wr = jnp.dot(rtab[...], w1[96:112, :], preferred_element_type=jnp.float32)
        h = jnp.dot(idemb[...], w1[0:64, :], preferred_element_type=jnp.float32)
        h = h + jnp.dot(oh_s, ws, preferred_element_type=jnp.float32)
        h = h + jnp.dot(oh_t, wt, preferred_element_type=jnp.float32)
        h = h + jnp.dot(oh_r, wr, preferred_element_type=jnp.float32)
        h = h + jnp.dot(num[...], w1[112:120, :], preferred_element_type=jnp.float32)
        h = jnp.maximum(h + b1[...], 0.0)
        h1b[pl.ds(base, CH), :] = h

        @pl.when(j == 0)
        def _():
            s1[...] = jnp.zeros_like(s1)
            q1[...] = jnp.zeros_like(q1)

        s1[...] += jnp.sum(h, axis=0, keepdims=True)
        q1[...] += jnp.sum(h * h, axis=0, keepdims=True)

    @pl.when(p == 1)
    def _phase1():
        mu = s1[...] * (1.0 / B)
        var = q1[...] * (1.0 / B) - mu * mu
        inv = g1[...] / jnp.sqrt(var + EPS)
        h = h1b[pl.ds(base, CH), :]
        h = (h - mu) * inv + be1[...]
        h = jnp.dot(h, w2[...], preferred_element_type=jnp.float32)
        h = jnp.maximum(h + b2[...], 0.0)
        h2b[pl.ds(base, CH), :] = h

        @pl.when(j == 0)
        def _():
            s2[...] = jnp.zeros_like(s2)
            q2[...] = jnp.zeros_like(q2)

        s2[...] += jnp.sum(h, axis=0, keepdims=True)
        q2[...] += jnp.sum(h * h, axis=0, keepdims=True)

    @pl.when(p == 2)
    def _phase2():
        mu = s2[...] * (1.0 / B)
        var = q2[...] * (1.0 / B) - mu * mu
        inv = g2[...] / jnp.sqrt(var + EPS)
        h = h2b[pl.ds(base, CH), :]
        h = (h - mu) * inv + be2[...]
        nrm = jnp.sqrt(jnp.sum(h * h, axis=1, keepdims=True))
        out[...] = h / jnp.maximum(nrm, 1e-12)


def _mlp_call(idemb, idx3, num, stab, ttab, rtabp, w1p,
              b1r, g1r, be1r, w2, b2r, g2r, be2r):
    chunk = lambda p, j: (j, 0)
    whole = lambda p, j: (0, 0)
    return pl.pallas_call(
        _mlp_body,
        grid=(3, NCH),
        in_specs=[
            pl.BlockSpec((CH, EMB), chunk),
            pl.BlockSpec((CH, 4), chunk),
            pl.BlockSpec((CH, 8), chunk),
            pl.BlockSpec((32, 16), whole),
            pl.BlockSpec((16, 16), whole),
            pl.BlockSpec((32, 16), whole),
            pl.BlockSpec((128, H1), whole),
            pl.BlockSpec((1, H1), whole),
            pl.BlockSpec((1, H1), whole),
            pl.BlockSpec((1, H1), whole),
            pl.BlockSpec((H1, H2), whole),
            pl.BlockSpec((1, H2), whole),
            pl.BlockSpec((1, H2), whole),
            pl.BlockSpec((1, H2), whole),
        ],
        out_specs=pl.BlockSpec((CH, H2), chunk),
        out_shape=jax.ShapeDtypeStruct((B, H2), jnp.float32),
        scratch_shapes=[
            pltpu.VMEM((B, H1), jnp.float32),
            pltpu.VMEM((B, H2), jnp.float32),
            pltpu.VMEM((1, H1), jnp.float32),
            pltpu.VMEM((1, H1), jnp.float32),
            pltpu.VMEM((1, H2), jnp.float32),
            pltpu.VMEM((1, H2), jnp.float32),
        ],
    )(idemb, idx3, num, stab, ttab, rtabp, w1p,
      b1r, g1r, be1r, w2, b2r, g2r, be2r)


def kernel(id, sector, stage, region, deal_size, revenue_multiple,
           growth_rate, profitability, team_experience, market_size,
           deal_table, sector_table, stage_table, region_table,
           W1, b1, g1, beta1, W2, b2, g2, beta2):
    idemb = _gather_deal(deal_table, id)
    zeros_i = jnp.zeros_like(sector)
    idx3 = jnp.stack([sector, stage, region, zeros_i], axis=-1)
    zeros_f = jnp.zeros_like(deal_size)
    num = jnp.stack([deal_size, revenue_multiple, growth_rate, profitability,
                     team_experience, market_size, zeros_f, zeros_f], axis=-1)
    w1p = jnp.pad(W1, ((0, 128 - W1.shape[0]), (0, 0)))
    rtabp = jnp.pad(region_table, ((0, 8), (0, 0)))
    return _mlp_call(
        idemb, idx3, num, sector_table, stage_table, rtabp, w1p,
        b1.reshape(1, H1), g1.reshape(1, H1), beta1.reshape(1, H1),
        W2, b2.reshape(1, H2), g2.reshape(1, H2), beta2.reshape(1, H2))


# speedup vs baseline: 1.1040x; 1.0578x over previous
import functools

import jax
import jax.numpy as jnp
from jax import lax
from jax.experimental import pallas as pl
from jax.experimental.pallas import tpu as pltpu
from jax.experimental.pallas import tpu_sc as plsc

B = 4096
EMB = 64
H1 = 256
H2 = 128
NW = 32
BPW = B // NW
CH = 512
NCH = B // CH
EPS = 1e-5


def _gather_deal(deal_table, idx):
    mesh = plsc.VectorSubcoreMesh(core_axis_name="c", subcore_axis_name="s")

    @functools.partial(
        pl.kernel,
        mesh=mesh,
        compiler_params=pltpu.CompilerParams(use_tc_tiling_on_sc=False),
        out_type=jax.ShapeDtypeStruct((B, EMB), jnp.float32),
        scratch_types=[
            pltpu.VMEM((BPW,), jnp.int32),
            pltpu.VMEM((BPW, EMB), jnp.float32),
            pltpu.SemaphoreType.DMA,
        ],
    )
    def gk(table_hbm, idx_hbm, out_hbm, idx_v, rows_v, sem):
        wid = lax.axis_index("s") * 2 + lax.axis_index("c")
        base = wid * BPW
        pltpu.sync_copy(idx_hbm.at[pl.ds(base, BPW)], idx_v)
        pltpu.async_copy(table_hbm.at[idx_v], rows_v, sem).wait()
        pltpu.sync_copy(rows_v, out_hbm.at[pl.ds(base, BPW)])

    return gk(deal_table, idx)


def _mlp_body(idemb, idx3, num, stab, ttab, rtab, w1, b1, g1, be1,
              w2, b2, g2, be2, out, h1b, h2b, s1, q1, s2, q2):
    p = pl.program_id(0)
    j = pl.program_id(1)
    base = j * CH
    ones_row = jnp.ones((8, CH), jnp.float32)

    @pl.when(p == 0)
    def _phase0():
        sec = idx3[:, 0:1]
        stg = idx3[:, 1:2]
        reg = idx3[:, 2:3]
        oh_s = (lax.broadcasted_iota(jnp.int32, (CH, 32), 1) == sec
                ).astype(jnp.float32)
        oh_t = (lax.broadcasted_iota(jnp.int32, (CH, 16), 1) == stg
                ).astype(jnp.float32)
        oh_r = (lax.broadcasted_iota(jnp.int32, (CH, 32), 1) == reg
                ).astype(jnp.float32)
        ws = jnp.dot(stab[...], w1[64:80, :], preferred_element_type=jnp.float32)
        wt = jnp.dot(ttab[...], w1[80:96, :], preferred_element_type=jnp.float32)
        wr = jnp.dot(rtab[...], w1[96:112, :], preferred_element_type=jnp.float32)
        h = jnp.dot(idemb[...], w1[0:64, :], preferred_element_type=jnp.float32)
        h = h + jnp.dot(oh_s, ws, preferred_element_type=jnp.float32)
        h = h + jnp.dot(oh_t, wt, preferred_element_type=jnp.float32)
        h = h + jnp.dot(oh_r, wr, preferred_element_type=jnp.float32)
        h = h + jnp.dot(num[...], w1[112:120, :], preferred_element_type=jnp.float32)
        h = jnp.maximum(h + b1[...], 0.0)
        h1b[pl.ds(base, CH), :] = h

        @pl.when(j == 0)
        def _():
            s1[...] = jnp.zeros_like(s1)
            q1[...] = jnp.zeros_like(q1)

        s1[...] += jnp.sum(h, axis=0, keepdims=True)
        q1[...] += jnp.sum(h * h, axis=0, keepdims=True)

    @pl.when(p == 1)
    def _phase1():
        mu = s1[...] * (1.0 / B)
        var = q1[...] * (1.0 / B) - mu * mu
        inv = g1[...] / jnp.sqrt(var + EPS)
        h = h1b[pl.ds(base, CH), :]
        h = (h - mu) * inv + be1[...]
        h = jnp.dot(h, w2[...], preferred_element_type=jnp.float32)
        h = jnp.maximum(h + b2[...], 0.0)
        h2b[pl.ds(base, CH), :] = h

        @pl.when(j == 0)
        def _():
            s2[...] = jnp.zeros_like(s2)
            q2[...] = jnp.zeros_like(q2)

        s2[...] += jnp.sum(h, axis=0, keepdims=True)
        q2[...] += jnp.sum(h * h, axis=0, keepdims=True)

    @pl.when(p == 2)
    def _phase2():
        mu = s2[...] * (1.0 / B)
        var = q2[...] * (1.0 / B) - mu * mu
        inv = g2[...] / jnp.sqrt(var + EPS)
        h = h2b[pl.ds(base, CH), :]
        h = (h - mu) * inv + be2[...]
        nrm = jnp.sqrt(jnp.sum(h * h, axis=1, keepdims=True))
        out[...] = h / jnp.maximum(nrm, 1e-12)


def _mlp_call(idemb, idx3, num, stab, ttab, rtabp, w1p,
              b1r, g1r, be1r, w2, b2r, g2r, be2r):
    chunk0 = lambda p, j: (jnp.where(p == 0, j, 0), 0)
    chunk2 = lambda p, j: (jnp.where(p == 2, j, 0), 0)
    whole = lambda p, j: (0, 0)
    return pl.pallas_call(
        _mlp_body,
        grid=(3, NCH),
        in_specs=[
            pl.BlockSpec((CH, EMB), chunk0),
            pl.BlockSpec((CH, 4), chunk0),
            pl.BlockSpec((CH, 8), chunk0),
            pl.BlockSpec((32, 16), whole),
            pl.BlockSpec((16, 16), whole),
            pl.BlockSpec((32, 16), whole),
            pl.BlockSpec((128, H1), whole),
            pl.BlockSpec((1, H1), whole),
            pl.BlockSpec((1, H1), whole),
            pl.BlockSpec((1, H1), whole),
            pl.BlockSpec((H1, H2), whole),
            pl.BlockSpec((1, H2), whole),
            pl.BlockSpec((1, H2), whole),
            pl.BlockSpec((1, H2), whole),
        ],
        out_specs=pl.BlockSpec((CH, H2), chunk2),
        out_shape=jax.ShapeDtypeStruct((B, H2), jnp.float32),
        scratch_shapes=[
            pltpu.VMEM((B, H1), jnp.float32),
            pltpu.VMEM((B, H2), jnp.float32),
            pltpu.VMEM((1, H1), jnp.float32),
            pltpu.VMEM((1, H1), jnp.float32),
            pltpu.VMEM((1, H2), jnp.float32),
            pltpu.VMEM((1, H2), jnp.float32),
        ],
    )(idemb, idx3, num, stab, ttab, rtabp, w1p,
      b1r, g1r, be1r, w2, b2r, g2r, be2r)


def kernel(id, sector, stage, region, deal_size, revenue_multiple,
           growth_rate, profitability, team_experience, market_size,
           deal_table, sector_table, stage_table, region_table,
           W1, b1, g1, beta1, W2, b2, g2, beta2):
    idemb = _gather_deal(deal_table, id)
    zeros_i = jnp.zeros_like(sector)
    idx3 = jnp.stack([sector, stage, region, zeros_i], axis=-1)
    zeros_f = jnp.zeros_like(deal_size)
    num = jnp.stack([deal_size, revenue_multiple, growth_rate, profitability,
                     team_experience, market_size, zeros_f, zeros_f], axis=-1)
    w1p = jnp.pad(W1, ((0, 128 - W1.shape[0]), (0, 0)))
    rtabp = jnp.pad(region_table, ((0, 8), (0, 0)))
    return _mlp_call(
        idemb, idx3, num, sector_table, stage_table, rtabp, w1p,
        b1.reshape(1, H1), g1.reshape(1, H1), beta1.reshape(1, H1),
        W2, b2.reshape(1, H2), g2.reshape(1, H2), beta2.reshape(1, H2))
